# baseline (speedup 1.0000x reference)
import functools

import jax
import jax.numpy as jnp
from jax import lax
from jax.experimental import pallas as pl
from jax.experimental.pallas import tpu as pltpu
from jax.experimental.pallas import tpu_sc as plsc

_INFO = plsc.get_sparse_core_info()
_NC = _INFO.num_cores
_NS = _INFO.num_subcores
_NW = _NC * _NS

_CHUNK = 512


@functools.lru_cache(maxsize=None)
def _make_gather(n_rows: int, dim: int):
    assert n_rows % (_NW * _CHUNK) == 0
    rows_per_w = n_rows // _NW
    n_chunks = rows_per_w // _CHUNK

    mesh = plsc.VectorSubcoreMesh(core_axis_name="c", subcore_axis_name="s")

    @functools.partial(
        pl.kernel,
        mesh=mesh,
        compiler_params=pltpu.CompilerParams(use_tc_tiling_on_sc=False),
        out_type=jax.ShapeDtypeStruct((n_rows, dim), jnp.float32),
        scratch_types=[
            pltpu.VMEM((_CHUNK,), jnp.int32),
            pltpu.VMEM((_CHUNK, dim), jnp.float32),
            pltpu.SemaphoreType.DMA,
        ],
    )
    def gather_kernel(idx_hbm, table_hbm, out_hbm, idx_v, rows_v, sem):
        wid = lax.axis_index("s") * _NC + lax.axis_index("c")
        base_w = wid * rows_per_w

        def body(i, carry):
            base = base_w + i * _CHUNK
            pltpu.sync_copy(idx_hbm.at[pl.ds(base, _CHUNK)], idx_v)
            pltpu.async_copy(table_hbm.at[idx_v], rows_v, sem).wait()
            pltpu.sync_copy(rows_v, out_hbm.at[pl.ds(base, _CHUNK)])
            return carry

        lax.fori_loop(0, n_chunks, body, 0)

    return gather_kernel


def kernel(inputs, table):
    batch, seq = inputs.shape
    _, dim = table.shape
    flat_idx = inputs.reshape(batch * seq).astype(jnp.int32)
    out = _make_gather(batch * seq, dim)(flat_idx, table)
    return out.reshape(batch, seq, dim)


# speedup vs baseline: 1.0475x; 1.0475x over previous
import functools

import jax
import jax.numpy as jnp
from jax import lax
from jax.experimental import pallas as pl
from jax.experimental.pallas import tpu as pltpu
from jax.experimental.pallas import tpu_sc as plsc

_INFO = plsc.get_sparse_core_info()
_NC = _INFO.num_cores
_NS = _INFO.num_subcores
_NW = _NC * _NS

_CHUNK = 512


@functools.lru_cache(maxsize=None)
def _make_gather(n_rows: int, dim: int):
    assert n_rows % (_NW * _CHUNK) == 0
    rows_per_w = n_rows // _NW
    n_chunks = rows_per_w // _CHUNK
    assert n_chunks % 2 == 0 and n_chunks >= 4

    mesh = plsc.VectorSubcoreMesh(core_axis_name="c", subcore_axis_name="s")

    @functools.partial(
        pl.kernel,
        mesh=mesh,
        compiler_params=pltpu.CompilerParams(use_tc_tiling_on_sc=False),
        out_type=jax.ShapeDtypeStruct((n_rows, dim), jnp.float32),
        scratch_types=[
            pltpu.VMEM((n_chunks, _CHUNK), jnp.int32),
            pltpu.VMEM((2, _CHUNK, dim), jnp.float32),
            pltpu.SemaphoreType.DMA,
            pltpu.SemaphoreType.DMA,
            pltpu.SemaphoreType.DMA,
            pltpu.SemaphoreType.DMA,
        ],
    )
    def gather_kernel(idx_hbm, table_hbm, out_hbm, idx_v, rows_v, sg0, sg1, so0, so1):
        wid = lax.axis_index("s") * _NC + lax.axis_index("c")
        base_w = wid * rows_per_w
        sg = (sg0, sg1)
        so = (so0, so1)

        def start_gather(i, s):
            pltpu.async_copy(table_hbm.at[idx_v.at[i]], rows_v.at[s], sg[s])

        def wait_gather(i, s):
            pltpu.make_async_copy(table_hbm.at[idx_v.at[i]], rows_v.at[s], sg[s]).wait()

        def start_out(i, s):
            pltpu.async_copy(rows_v.at[s], out_hbm.at[pl.ds(base_w + i * _CHUNK, _CHUNK)], so[s])

        def wait_out(i, s):
            pltpu.make_async_copy(
                rows_v.at[s], out_hbm.at[pl.ds(base_w + i * _CHUNK, _CHUNK)], so[s]
            ).wait()

        pltpu.sync_copy(idx_hbm.at[wid], idx_v)

        start_gather(0, 0)
        wait_gather(0, 0)
        start_gather(1, 1)
        start_out(0, 0)

        def pair_body(j, carry):
            for t in range(2):
                i = 2 * j + 1 + t
                s = 1 - t
                wait_gather(i, s)
                wait_out(i - 1, 1 - s)
                start_gather(i + 1, 1 - s)
                start_out(i, s)
            return carry

        lax.fori_loop(0, (n_chunks - 2) // 2, pair_body, 0)

        last = n_chunks - 1
        wait_gather(last, 1)
        wait_out(last - 1, 0)
        start_out(last, 1)
        wait_out(last, 1)

    return gather_kernel


def kernel(inputs, table):
    batch, seq = inputs.shape
    _, dim = table.shape
    n_rows = batch * seq
    rows_per_w = n_rows // _NW
    n_chunks = rows_per_w // _CHUNK
    flat_idx = inputs.reshape(_NW, n_chunks, _CHUNK).astype(jnp.int32)
    out = _make_gather(n_rows, dim)(flat_idx, table)
    return out.reshape(batch, seq, dim)
